# baseline (speedup 1.0000x reference)
import functools

import jax
import jax.numpy as jnp
from jax.experimental import pallas as pl

PROTO_NUM = 8192
PROTO_DIM = 256
T_TILE = 512
K_TILE = 1024
N_K = PROTO_NUM // K_TILE


def _vq_body(x_ref, xn_ref, pn_ref, p_ref, proto_ref, xout_ref, loss_ref):
    x = x_ref[...]
    xb = x.astype(jnp.bfloat16)
    xn = xn_ref[...]

    run_min = jnp.full((T_TILE, 1), jnp.inf, dtype=jnp.float32)
    run_idx = jnp.zeros((T_TILE, 1), dtype=jnp.int32)

    for g in range(N_K // 2):
        m2 = None
        i2 = None
        for h in range(2):
            c = g * 2 + h
            p = p_ref[c * K_TILE:(c + 1) * K_TILE, :]
            pn = pn_ref[:, c * K_TILE:(c + 1) * K_TILE]
            mm = jax.lax.dot_general(
                xb, p.astype(jnp.bfloat16), (((1,), (1,)), ((), ())),
                preferred_element_type=jnp.float32)
            s = (xn + pn) - 2.0 * mm
            m = jnp.min(s, axis=1, keepdims=True)
            cols = jax.lax.broadcasted_iota(
                jnp.int32, (T_TILE, K_TILE), 1) + c * K_TILE
            cand = jnp.where(s == m, cols, jnp.int32(2**30))
            li = jnp.min(cand, axis=1, keepdims=True)
            if m2 is None:
                m2, i2 = m, li
            else:
                sub_better = m < m2
                i2 = jnp.where(sub_better, li, i2)
                m2 = jnp.where(sub_better, m, m2)
        better = m2 < run_min
        run_idx = jnp.where(better, i2, run_idx)
        run_min = jnp.where(
            better, m2.astype(jnp.bfloat16).astype(jnp.float32), run_min)

    proto = jnp.zeros((T_TILE, PROTO_DIM), dtype=jnp.float32)
    for c in range(N_K):
        p = p_ref[c * K_TILE:(c + 1) * K_TILE, :]
        cols = jax.lax.broadcasted_iota(
            jnp.int32, (T_TILE, K_TILE), 1) + c * K_TILE
        oh = (cols == run_idx).astype(jnp.bfloat16)
        proto = proto + jax.lax.dot_general(
            oh, p.astype(jnp.bfloat16), (((1,), (0,)), ((), ())),
            preferred_element_type=jnp.float32)

    d = proto - x
    proto_st = x + d
    proto_ref[...] = proto_st
    xout_ref[...] = x - proto_st
    part = jnp.sum(d * d).reshape(1, 1)

    @pl.when(pl.program_id(0) == 0)
    def _init():
        loss_ref[...] = part

    @pl.when(pl.program_id(0) != 0)
    def _acc():
        loss_ref[...] = loss_ref[...] + part


@functools.partial(jax.jit, static_argnames=("interpret",))
def _vq(xf, xn, pn, prototypes, interpret=False):
    n_t = xf.shape[0] // T_TILE
    proto, x_out, loss_sum = pl.pallas_call(
        _vq_body,
        grid=(n_t,),
        in_specs=[
            pl.BlockSpec((T_TILE, PROTO_DIM), lambda i: (i, 0)),
            pl.BlockSpec((T_TILE, 1), lambda i: (i, 0)),
            pl.BlockSpec((1, PROTO_NUM), lambda i: (0, 0)),
            pl.BlockSpec((PROTO_NUM, PROTO_DIM), lambda i: (0, 0)),
        ],
        out_specs=[
            pl.BlockSpec((T_TILE, PROTO_DIM), lambda i: (i, 0)),
            pl.BlockSpec((T_TILE, PROTO_DIM), lambda i: (i, 0)),
            pl.BlockSpec((1, 1), lambda i: (0, 0)),
        ],
        out_shape=[
            jax.ShapeDtypeStruct(xf.shape, jnp.float32),
            jax.ShapeDtypeStruct(xf.shape, jnp.float32),
            jax.ShapeDtypeStruct((1, 1), jnp.float32),
        ],
        interpret=interpret,
    )(xf, xn, pn, prototypes)
    return proto, x_out, loss_sum


def kernel(x, prototypes):
    x_shape = x.shape
    xf = x.reshape(-1, PROTO_DIM)
    xn = jnp.sum(x ** 2, axis=-1).reshape(-1, 1)
    pn = jnp.sum(prototypes ** 2, axis=1).reshape(1, PROTO_NUM)
    proto_st, x_out, loss_sum = _vq(xf, xn, pn, prototypes)
    q = loss_sum[0, 0] / jnp.float32(xf.size)
    loss = q + 0.25 * q
    return (proto_st.reshape(x_shape), x_out.reshape(x_shape), loss)
